# Initial kernel scaffold; baseline (speedup 1.0000x reference)
import jax
import jax.numpy as jnp
from jax.experimental import pallas as pl


def kernel(num_samples, weights, uniforms):
    raise NotImplementedError("write your pallas kernel here")


# speedup vs baseline: 102.7049x; 102.7049x over previous
import functools

import jax
import jax.numpy as jnp
from jax import lax
from jax.experimental import pallas as pl
from jax.experimental.pallas import tpu as pltpu
from jax.experimental.pallas import tpu_sc as plsc

D = 4096
K = 32
BIG_NEG = -1e30
PROW = 34
T_CHUNK = 1024
N_CHUNK = D // T_CHUNK


def _dp_body(w_ref, pt_ref, srow_ref, buf_ref):
    lane = lax.broadcasted_iota(jnp.int32, (1, 128), 1)
    neg = jnp.full((1, 1), BIG_NEG, dtype=jnp.float32)

    def outer(i, s):
        for k in range(8):
            m = i * 8 + k
            w = w_ref[m]
            shifted = jnp.concatenate([neg, s[:, :-1]], axis=1)
            t1 = shifted + w
            amax = jnp.maximum(t1, s)
            s_new = amax + jnp.log1p(jnp.exp(-jnp.abs(t1 - s)))
            p = jnp.exp(jnp.minimum(t1 - s_new, 30.0))
            p = jnp.where(lane >= m + 1, 1.0, p)
            p = jnp.where(lane == 0, 0.0, p)
            p = jnp.where(lane == 33, w, p)
            buf_ref[7 - k : 8 - k, :] = p
            s = s_new
        pt_ref[pl.ds(D - 8 - i * 8, 8), :] = buf_ref[:, :]
        return s

    s0 = jnp.where(lane == 0, 0.0, jnp.full((1, 128), BIG_NEG, jnp.float32))
    s = lax.fori_loop(0, D // 8, outer, s0)
    srow_ref[0:1, :] = s


def _dp_tables(weights):
    return pl.pallas_call(
        _dp_body,
        in_specs=[pl.BlockSpec(memory_space=pltpu.SMEM)],
        out_shape=[
            jax.ShapeDtypeStruct((D, 128), jnp.float32),
            jax.ShapeDtypeStruct((8, 128), jnp.float32),
        ],
        scratch_shapes=[pltpu.VMEM((8, 128), jnp.float32)],
    )(weights)


def _sampler_body(nds_hbm, pt_hbm, uni_hbm, out_s, out_lp,
                  pt_v, u_v, s_v, nds_v, lp_v):
    wid = lax.axis_index("c") * 16 + lax.axis_index("s")

    @pl.when(wid < 8)
    def _():
        base = wid * 16
        pltpu.sync_copy(nds_hbm, nds_v)
        ndts0 = nds_v[...]
        zeros_i = jnp.full((16,), 0, jnp.int32)
        c33 = jnp.full((16,), 33, jnp.int32)
        lp0 = jnp.full((16,), 0.0, jnp.float32)

        def chunk(c, carry):
            ndts, lp = carry
            pltpu.sync_copy(pt_hbm.at[pl.ds(c * T_CHUNK * PROW, T_CHUNK * PROW)], pt_v)
            pltpu.sync_copy(uni_hbm.at[pl.ds(c * T_CHUNK, T_CHUNK), pl.ds(base, 16)], u_v)

            def step(t, carry2):
                ndts, lp, bvec = carry2
                p = plsc.load_gather(pt_v, [bvec + ndts])
                wv = plsc.load_gather(pt_v, [bvec + c33])
                u = u_v[t]
                sm = u < p
                s_i = jnp.where(sm, 1, 0)
                s_v[t] = s_i
                ndts = ndts - s_i
                lp = lp + jnp.where(sm, wv, 0.0)
                return ndts, lp, bvec + PROW

            ndts, lp, _ = lax.fori_loop(0, T_CHUNK, step, (ndts, lp, zeros_i))
            pltpu.sync_copy(s_v, out_s.at[pl.ds(c * T_CHUNK, T_CHUNK), pl.ds(base, 16)])
            return ndts, lp

        carry = (ndts0, lp0)
        for c in range(N_CHUNK):
            carry = chunk(c, carry)
        lp_v[...] = carry[1]
        pltpu.sync_copy(lp_v, out_lp.at[pl.ds(base, 16)])


@functools.cache
def _get_sampler():
    return functools.partial(
        pl.kernel,
        out_type=(
            jax.ShapeDtypeStruct((D, 128), jnp.int32),
            jax.ShapeDtypeStruct((128,), jnp.float32),
        ),
        mesh=plsc.VectorSubcoreMesh(
            core_axis_name="c", subcore_axis_name="s", num_cores=2, num_subcores=16
        ),
        compiler_params=pltpu.CompilerParams(
            use_tc_tiling_on_sc=False, needs_layout_passes=False
        ),
        scratch_types=[
            pltpu.VMEM((T_CHUNK * PROW,), jnp.float32),
            pltpu.VMEM((T_CHUNK, 16), jnp.float32),
            pltpu.VMEM((T_CHUNK, 16), jnp.int32),
            pltpu.VMEM((16,), jnp.int32),
            pltpu.VMEM((16,), jnp.float32),
        ],
    )(_sampler_body)


def kernel(num_samples, weights, uniforms):
    n = uniforms.shape[1]
    ptable, srow = _dp_tables(weights)
    logz = srow[0, K]
    pt_flat = ptable[:, :PROW].reshape(-1)
    nds0 = jnp.full(
        (16,), K + (jnp.asarray(num_samples, jnp.int32) - n), dtype=jnp.int32
    )
    ys, lp = _get_sampler()(nds0, pt_flat, uniforms)
    samples = ys[::-1].T
    logprob = lp - logz
    return samples, logprob
